# Initial kernel scaffold; baseline (speedup 1.0000x reference)
import jax
import jax.numpy as jnp
from jax.experimental import pallas as pl


def kernel(x, edge_index, edge_attr, u, params):
    raise NotImplementedError("write your pallas kernel here")


# speedup vs baseline: 3.0578x; 3.0578x over previous
import functools

import jax
import jax.numpy as jnp
from jax import lax
from jax.experimental import pallas as pl
from jax.experimental.pallas import tpu as pltpu
from jax.experimental.pallas import tpu_sc as plsc

N_E = 3_200_000
N_N = 100_000
R = 125
NROWS = N_E // R
NW = 32
ROWS_W = NROWS // NW
NTILE = 16
NODES_T = N_N // NTILE
M_E = 6400
M_N = 5000
_OUT_ACT = [True, True, True, True, True, False]
_INTERPRET = False


def _leaky(h):
    return jnp.where(h >= 0, h, 0.01 * h)


@functools.partial(jax.jit, static_argnums=(3,))
def _sc_gather(xpad, src, dst, dnp):
    K = 8 if dnp > 16 else 16
    nch = ROWS_W // K
    mesh = plsc.VectorSubcoreMesh(core_axis_name="c", subcore_axis_name="s")

    def body(x_hbm, src_hbm, dst_hbm, xs_hbm, xd_hbm,
             idx_s, idx_d, rows_s, rows_d, sem):
        cid = lax.axis_index("c")
        sid = lax.axis_index("s")
        wid = sid * 2 + cid
        base = wid * ROWS_W

        def chunk(ci, carry):
            r0 = base + ci * K
            pltpu.sync_copy(src_hbm.at[pl.ds(r0, K)], idx_s)
            pltpu.sync_copy(dst_hbm.at[pl.ds(r0, K)], idx_d)
            cps = []
            for j in range(K):
                cps.append(pltpu.async_copy(x_hbm.at[idx_s.at[j]], rows_s.at[j], sem))
                cps.append(pltpu.async_copy(x_hbm.at[idx_d.at[j]], rows_d.at[j], sem))
            for c in cps:
                c.wait()
            pltpu.sync_copy(rows_s, xs_hbm.at[pl.ds(r0, K)])
            pltpu.sync_copy(rows_d, xd_hbm.at[pl.ds(r0, K)])
            return carry

        lax.fori_loop(0, nch, chunk, 0)

    fn = pl.kernel(
        body,
        out_type=[jax.ShapeDtypeStruct((NROWS, R, dnp), jnp.float32),
                  jax.ShapeDtypeStruct((NROWS, R, dnp), jnp.float32)],
        mesh=mesh,
        scratch_types=[pltpu.VMEM((K, R), jnp.int32),
                       pltpu.VMEM((K, R), jnp.int32),
                       pltpu.VMEM((K, R, dnp), jnp.float32),
                       pltpu.VMEM((K, R, dnp), jnp.float32),
                       pltpu.SemaphoreType.DMA],
        compiler_params=pltpu.CompilerParams(use_tc_tiling_on_sc=False),
    )
    return fn(xpad, src, dst)


@functools.partial(jax.jit, static_argnums=(3,))
def _sc_scatter(eor, dst, zeros, dg):
    K = 16
    nch = ROWS_W // K
    mesh = plsc.VectorSubcoreMesh(core_axis_name="c", subcore_axis_name="s")

    def body(eor_hbm, dst_hbm, zeros_hbm, agg_hbm, idx_d, rows, table, sem):
        cid = lax.axis_index("c")
        sid = lax.axis_index("s")
        pltpu.sync_copy(zeros_hbm, table.at[pl.ds(sid * NODES_T, NODES_T)])
        plsc.subcore_barrier()
        wid = sid * 2 + cid
        base = wid * ROWS_W

        def chunk(ci, carry):
            r0 = base + ci * K
            pltpu.sync_copy(dst_hbm.at[pl.ds(r0, K)], idx_d)
            pltpu.sync_copy(eor_hbm.at[pl.ds(r0, K)], rows)
            cps = []
            for j in range(K):
                cps.append(pltpu.async_copy(
                    rows.at[j], table.at[idx_d.at[j]], sem, add=True))
            for c in cps:
                c.wait()
            return carry

        lax.fori_loop(0, nch, chunk, 0)
        plsc.subcore_barrier()
        pltpu.sync_copy(table.at[pl.ds(sid * NODES_T, NODES_T)],
                        agg_hbm.at[pl.ds(cid * N_N + sid * NODES_T, NODES_T)])

    fn = pl.kernel(
        body,
        out_type=jax.ShapeDtypeStruct((2 * N_N, dg), jnp.float32),
        mesh=mesh,
        scratch_types=[pltpu.VMEM((K, R), jnp.int32),
                       pltpu.VMEM((K, R, dg), jnp.float32),
                       pltpu.VMEM_SHARED((N_N, dg), jnp.float32),
                       pltpu.SemaphoreType.DMA],
        compiler_params=pltpu.CompilerParams(use_tc_tiling_on_sc=False),
    )
    return fn(eor, dst, zeros)


def _edge_mlp(xs, xd, e_groups, u, layers, dn, dnp, out_groups, residual, out_act):
    M = M_E
    grid = N_E // M
    du = u.shape[1]
    W1, b1 = layers[0]
    A = W1[0:dn]
    B = W1[dn:2 * dn]
    off = 2 * dn
    Cs = []
    for eg in e_groups:
        w = eg.shape[1]
        Cs.append(W1[off:off + w])
        off += w
    D = W1[off:off + du]
    if dnp != dn:
        A = jnp.pad(A, ((0, dnp - dn), (0, 0)))
        B = jnp.pad(B, ((0, dnp - dn), (0, 0)))
    rest = layers[1:]
    n_eg = len(e_groups)
    n_rest = len(rest)
    n_og = len(out_groups)

    def body(*refs):
        it = iter(refs)
        xs_r = next(it)
        xd_r = next(it)
        e_rs = [next(it) for _ in range(n_eg)]
        u_r = next(it)
        A_r = next(it)
        B_r = next(it)
        C_rs = [next(it) for _ in range(n_eg)]
        D_r = next(it)
        b1_r = next(it)
        rest_rs = [(next(it), next(it)) for _ in range(n_rest)]
        out_rs = [next(it) for _ in range(n_og)]
        esum_rs = [next(it) for _ in range(n_og)]

        h = jnp.dot(xs_r[...], A_r[...], preferred_element_type=jnp.float32)
        h = h + jnp.dot(xd_r[...], B_r[...], preferred_element_type=jnp.float32)
        for e_r, C_r in zip(e_rs, C_rs):
            h = h + jnp.dot(e_r[...], C_r[...], preferred_element_type=jnp.float32)
        h = h + (jnp.dot(u_r[...], D_r[...], preferred_element_type=jnp.float32)
                 + b1_r[...])
        h = _leaky(h)
        for k, (W_r, b_r) in enumerate(rest_rs):
            h = jnp.dot(h, W_r[...], preferred_element_type=jnp.float32) + b_r[...]
            if k < n_rest - 1:
                h = _leaky(h)
        if out_act:
            h = _leaky(h)
        if residual:
            e_full = (e_rs[0][...] if n_eg == 1 else
                      jnp.concatenate([r[...] for r in e_rs], axis=1))
            h = h + e_full

        def store(gi, g):
            @pl.when(pl.program_id(0) == 0)
            def _():
                esum_rs[gi][...] = jnp.zeros_like(esum_rs[gi])
            out_rs[gi][...] = g
            esum_rs[gi][...] += jnp.sum(g, axis=0, keepdims=True)

        off2 = 0
        for gi, w in enumerate(out_groups):
            store(gi, h[:, off2:off2 + w])
            off2 += w

    def row_map(i):
        return (i, 0)

    def fix_map(i):
        return (0, 0)

    biases = [b1.reshape(1, -1)] + [b.reshape(1, -1) for _, b in rest]
    in_arrays = ([xs, xd] + list(e_groups) + [u, A, B] + Cs + [D, biases[0]])
    in_specs = ([pl.BlockSpec((M, dnp), row_map), pl.BlockSpec((M, dnp), row_map)]
                + [pl.BlockSpec((M, eg.shape[1]), row_map) for eg in e_groups]
                + [pl.BlockSpec(u.shape, fix_map),
                   pl.BlockSpec(A.shape, fix_map),
                   pl.BlockSpec(B.shape, fix_map)]
                + [pl.BlockSpec(c.shape, fix_map) for c in Cs]
                + [pl.BlockSpec(D.shape, fix_map),
                   pl.BlockSpec(biases[0].shape, fix_map)])
    for k, (W, _) in enumerate(rest):
        in_arrays += [W, biases[k + 1]]
        in_specs += [pl.BlockSpec(W.shape, fix_map),
                     pl.BlockSpec(biases[k + 1].shape, fix_map)]

    out_shape = ([jax.ShapeDtypeStruct((N_E, w), jnp.float32) for w in out_groups]
                 + [jax.ShapeDtypeStruct((1, w), jnp.float32) for w in out_groups])
    out_specs = ([pl.BlockSpec((M, w), row_map) for w in out_groups]
                 + [pl.BlockSpec((1, w), fix_map) for w in out_groups])

    res = pl.pallas_call(
        body,
        grid=(grid,),
        in_specs=in_specs,
        out_specs=out_specs,
        out_shape=out_shape,
        compiler_params=pltpu.CompilerParams(
            dimension_semantics=("arbitrary",)),
        interpret=_INTERPRET,
    )(*in_arrays)
    return res[:n_og], res[n_og:]


def _node_mlp(x, agg_parts, u, layers, residual, out_act):
    Mn = M_N
    grid = N_N // Mn
    dn = x.shape[1]
    du = u.shape[1]
    W1, b1 = layers[0]
    A = W1[0:dn]
    off = dn
    Bs = []
    for p in agg_parts:
        w = p.shape[2]
        Bs.append(W1[off:off + w])
        off += w
    C = W1[off:off + du]
    rest = layers[1:]
    n_g = len(agg_parts)
    n_rest = len(rest)

    def body(*refs):
        it = iter(refs)
        x_r = next(it)
        p_rs = [next(it) for _ in range(n_g)]
        u_r = next(it)
        A_r = next(it)
        B_rs = [next(it) for _ in range(n_g)]
        C_r = next(it)
        b1_r = next(it)
        rest_rs = [(next(it), next(it)) for _ in range(n_rest)]
        out_r = next(it)
        xsum_r = next(it)

        h = jnp.dot(x_r[...], A_r[...], preferred_element_type=jnp.float32)
        for p_r, B_r in zip(p_rs, B_rs):
            agg = p_r[0] + p_r[1]
            h = h + jnp.dot(agg, B_r[...], preferred_element_type=jnp.float32)
        h = h + (jnp.dot(u_r[...], C_r[...], preferred_element_type=jnp.float32)
                 + b1_r[...])
        h = _leaky(h)
        for k, (W_r, b_r) in enumerate(rest_rs):
            h = jnp.dot(h, W_r[...], preferred_element_type=jnp.float32) + b_r[...]
            if k < n_rest - 1:
                h = _leaky(h)
        if out_act:
            h = _leaky(h)
        if residual:
            h = h + x_r[...]
        out_r[...] = h

        @pl.when(pl.program_id(0) == 0)
        def _():
            xsum_r[...] = jnp.zeros_like(xsum_r)
        xsum_r[...] += jnp.sum(h, axis=0, keepdims=True)

    def row_map(i):
        return (i, 0)

    def fix_map(i):
        return (0, 0)

    def part_map(i):
        return (0, i, 0)

    biases = [b1.reshape(1, -1)] + [b.reshape(1, -1) for _, b in rest]
    parts3 = [p.reshape(2, N_N, p.shape[2]) for p in agg_parts]
    in_arrays = [x] + parts3 + [u, A] + Bs + [C, biases[0]]
    in_specs = ([pl.BlockSpec((Mn, dn), row_map)]
                + [pl.BlockSpec((2, Mn, p.shape[2]), part_map) for p in parts3]
                + [pl.BlockSpec(u.shape, fix_map),
                   pl.BlockSpec(A.shape, fix_map)]
                + [pl.BlockSpec(b.shape, fix_map) for b in Bs]
                + [pl.BlockSpec(C.shape, fix_map),
                   pl.BlockSpec(biases[0].shape, fix_map)])
    for k, (W, _) in enumerate(rest):
        in_arrays += [W, biases[k + 1]]
        in_specs += [pl.BlockSpec(W.shape, fix_map),
                     pl.BlockSpec(biases[k + 1].shape, fix_map)]

    d_out = rest[-1][0].shape[1]
    out_shape = [jax.ShapeDtypeStruct((N_N, d_out), jnp.float32),
                 jax.ShapeDtypeStruct((1, d_out), jnp.float32)]
    out_specs = [pl.BlockSpec((Mn, d_out), row_map),
                 pl.BlockSpec((1, d_out), fix_map)]

    x_out, xsum = pl.pallas_call(
        body,
        grid=(grid,),
        in_specs=in_specs,
        out_specs=out_specs,
        out_shape=out_shape,
        compiler_params=pltpu.CompilerParams(
            dimension_semantics=("arbitrary",)),
        interpret=_INTERPRET,
    )(*in_arrays)
    return x_out, xsum


def _glob_mlp(esums, xsum, u, layers, residual, out_act):
    W1, b1 = layers[0]
    widths = [e.shape[1] for e in esums] + [xsum.shape[1], u.shape[1]]
    splits = []
    off = 0
    for w in widths:
        splits.append(W1[off:off + w])
        off += w
    rest = layers[1:]
    n_in = len(esums) + 2
    n_rest = len(rest)

    def body(*refs):
        it = iter(refs)
        in_rs = [next(it) for _ in range(n_in)]
        s_rs = [next(it) for _ in range(n_in)]
        b1_r = next(it)
        rest_rs = [(next(it), next(it)) for _ in range(n_rest)]
        out_r = next(it)

        h = b1_r[...]
        for in_r, s_r in zip(in_rs, s_rs):
            h = h + jnp.dot(in_r[...], s_r[...], preferred_element_type=jnp.float32)
        h = _leaky(h)
        for k, (W_r, b_r) in enumerate(rest_rs):
            h = jnp.dot(h, W_r[...], preferred_element_type=jnp.float32) + b_r[...]
            if k < n_rest - 1:
                h = _leaky(h)
        if out_act:
            h = _leaky(h)
        if residual:
            h = h + in_rs[-1][...]
        out_r[...] = h

    biases = [b1.reshape(1, -1)] + [b.reshape(1, -1) for _, b in rest]
    in_arrays = list(esums) + [xsum, u] + splits + [biases[0]]
    for k, (W, _) in enumerate(rest):
        in_arrays += [W, biases[k + 1]]

    d_out = rest[-1][0].shape[1]
    u_out = pl.pallas_call(
        body,
        out_shape=jax.ShapeDtypeStruct((1, d_out), jnp.float32),
        interpret=_INTERPRET,
    )(*in_arrays)
    return u_out


def kernel(x, edge_index, edge_attr, u, params):
    src = edge_index[0].astype(jnp.int32).reshape(NROWS, R)
    dst = edge_index[1].astype(jnp.int32).reshape(NROWS, R)
    e_groups = [edge_attr]
    xg = x
    ug = u
    for i, layer in enumerate(params):
        out_act = _OUT_ACT[i]
        dn = xg.shape[1]
        dnp = ((dn + 3) // 4) * 4
        xpad = xg if dnp == dn else jnp.pad(xg, ((0, 0), (0, dnp - dn)))
        xs3, xd3 = _sc_gather(xpad, src, dst, dnp)
        xs = xs3.reshape(N_E, dnp)
        xd = xd3.reshape(N_E, dnp)

        eW = layer['edge']
        do_e = eW[-1][0].shape[1]
        de_in = sum(eg.shape[1] for eg in e_groups)
        nW = layer['node']
        do_n = nW[-1][0].shape[1]
        residual = (de_in == do_e) and (dn == do_n)
        out_groups = [8] * (do_e // 8) if do_e > 8 else [do_e]

        e_outs, esums = _edge_mlp(xs, xd, e_groups, ug, eW, dn, dnp,
                                  out_groups, residual, out_act)

        agg_parts = []
        for w, eo in zip(out_groups, e_outs):
            eor = eo.reshape(NROWS, R, w)
            zeros = jnp.zeros((NODES_T, w), jnp.float32)
            agg_parts.append(_sc_scatter(eor, dst, zeros, w).reshape(2, N_N, w))

        x_out, xsum = _node_mlp(xg, agg_parts, ug, nW, residual, out_act)
        u_out = _glob_mlp(esums, xsum, ug, layer['glob'], residual, out_act)

        xg, e_groups, ug = x_out, list(e_outs), u_out

    e_final = e_groups[0] if len(e_groups) == 1 else jnp.concatenate(e_groups, axis=1)
    return xg, e_final, ug
